# baseline (speedup 1.0000x reference)
import jax
import jax.numpy as jnp
from jax import lax
from jax.experimental import pallas as pl
from jax.experimental.pallas import tpu as pltpu
from jax.experimental.pallas import tpu_sc as plsc

B, T, V, D, H, F = 1024, 50, 1000, 512, 512, 2048
G = 4 * H
GP = G // 2
OUT_PAD = 128

TCH = 5
NCH = T // TCH

_NC, _NS = 2, 16
_NW = _NC * _NS
_ROWS_PER_W = (TCH * B) // _NW
_CHUNK = 80
_NCHUNK = _ROWS_PER_W // _CHUNK


def _proj_body(emb_ref, w_ref, b_ref, out_ref):
    p = (
        jnp.dot(emb_ref[...], w_ref[...], preferred_element_type=jnp.float32)
        + b_ref[...]
    )
    lo_bits = lax.bitcast_convert_type(p[:, :GP], jnp.int32) + 0x8000
    hi_bits = lax.bitcast_convert_type(p[:, GP:], jnp.int32) + 0x8000
    lo16 = lax.shift_right_logical(lo_bits, 16)
    hi16 = hi_bits & jnp.int32(-65536)
    out_ref[...] = hi16 | lo16


def _gather_body(table_hbm, idx_hbm, out_hbm, idx_v, rows_v, sem):
    wid = lax.axis_index("s") * _NC + lax.axis_index("c")
    base = wid * _ROWS_PER_W
    pltpu.sync_copy(idx_hbm.at[pl.ds(base, _ROWS_PER_W)], idx_v)

    def chunk(ch, carry):
        r0 = ch * _CHUNK
        pltpu.async_copy(
            table_hbm.at[idx_v.at[pl.ds(r0, _CHUNK)]], rows_v, sem
        ).wait()
        pltpu.sync_copy(rows_v, out_hbm.at[pl.ds(base + r0, _CHUNK)])
        return carry

    lax.fori_loop(0, _NCHUNK, chunk, 0)


def _lstm_body(x_ref, whh_ref, h_in_ref, c_in_ref, h_ref, c_ref):
    t = pl.program_id(0)

    @pl.when(t == 0)
    def _():
        h_ref[...] = h_in_ref[...]
        c_ref[...] = c_in_ref[...]

    w = x_ref[0]
    x_lo = lax.bitcast_convert_type(lax.shift_left(w, 16), jnp.float32)
    x_hi = lax.bitcast_convert_type(w & jnp.int32(-65536), jnp.float32)
    hw = jnp.dot(h_ref[...], whh_ref[...], preferred_element_type=jnp.float32)
    i = jax.nn.sigmoid(x_lo[:, 0:H] + hw[:, 0:H])
    f = jax.nn.sigmoid(x_lo[:, H:2 * H] + hw[:, H:2 * H])
    g = jnp.tanh(x_hi[:, 0:H] + hw[:, 2 * H:3 * H])
    o = jax.nn.sigmoid(x_hi[:, H:2 * H] + hw[:, 3 * H:4 * H])
    c_new = f * c_ref[...] + i * g
    h_new = o * jnp.tanh(c_new)
    c_ref[...] = c_new
    h_ref[...] = h_new.astype(jnp.bfloat16)


def _mlp_body(h_ref, w1_ref, b1_ref, w2_ref, b2_ref, out_ref):
    a = jnp.maximum(
        jnp.dot(h_ref[...], w1_ref[...], preferred_element_type=jnp.float32)
        + b1_ref[...],
        0.0,
    ).astype(jnp.bfloat16)
    out_ref[...] = (
        jnp.dot(a, w2_ref[...], preferred_element_type=jnp.float32)
        + b2_ref[...]
    )


def kernel(src_seq, src_pos, emb, W_ih, W_hh, b_ih, b_hh, W1, b1, W2, b2):
    bias = (b_ih + b_hh).reshape(1, G)
    P_packed = pl.pallas_call(
        _proj_body,
        out_shape=jax.ShapeDtypeStruct((V, GP), jnp.int32),
    )(emb, W_ih.T, bias)

    flat_idx = src_seq.T.reshape(NCH, TCH * B).astype(jnp.int32)
    gather = pl.kernel(
        _gather_body,
        out_type=jax.ShapeDtypeStruct((TCH * B, GP), jnp.int32),
        mesh=plsc.VectorSubcoreMesh(core_axis_name="c", subcore_axis_name="s"),
        scratch_types=[
            pltpu.VMEM((_ROWS_PER_W,), jnp.int32),
            pltpu.VMEM((_CHUNK, GP), jnp.int32),
            pltpu.SemaphoreType.DMA,
        ],
    )

    whh_bf = W_hh.T.astype(jnp.bfloat16)
    lstm_chunk = pl.pallas_call(
        _lstm_body,
        grid=(TCH,),
        in_specs=[
            pl.BlockSpec((1, B, GP), lambda t: (t, 0, 0)),
            pl.BlockSpec((H, G), lambda t: (0, 0)),
            pl.BlockSpec((B, H), lambda t: (0, 0)),
            pl.BlockSpec((B, H), lambda t: (0, 0)),
        ],
        out_specs=[
            pl.BlockSpec((B, H), lambda t: (0, 0)),
            pl.BlockSpec((B, H), lambda t: (0, 0)),
        ],
        out_shape=[
            jax.ShapeDtypeStruct((B, H), jnp.bfloat16),
            jax.ShapeDtypeStruct((B, H), jnp.float32),
        ],
    )

    h = jnp.zeros((B, H), jnp.bfloat16)
    c = jnp.zeros((B, H), jnp.float32)
    for k in range(NCH):
        X_k = gather(P_packed, flat_idx[k]).reshape(TCH, B, GP)
        h, c = lstm_chunk(X_k, whh_bf, h, c)

    W2p = jnp.pad(W2.T, ((0, 0), (0, OUT_PAD - 2))).astype(jnp.bfloat16)
    b2p = jnp.pad(b2, (0, OUT_PAD - 2)).reshape(1, OUT_PAD)
    out_p = pl.pallas_call(
        _mlp_body,
        out_shape=jax.ShapeDtypeStruct((B, OUT_PAD), jnp.float32),
    )(h, W1.T.astype(jnp.bfloat16), b1.reshape(1, F), W2p, b2p)
    return out_p[:, :2]


# speedup vs baseline: 1.0720x; 1.0720x over previous
import jax
import jax.numpy as jnp
from jax import lax
from jax.experimental import pallas as pl
from jax.experimental.pallas import tpu as pltpu
from jax.experimental.pallas import tpu_sc as plsc

B, T, V, D, H, F = 1024, 50, 1000, 512, 512, 2048
G = 4 * H
GP = G // 2
OUT_PAD = 128

CH_SIZES = (12, 12, 12, 12, 2)

_NC, _NS = 2, 16
_NW = _NC * _NS
_CHUNK = 64


def _proj_body(emb_ref, w_ref, b_ref, out_ref):
    p = (
        jnp.dot(emb_ref[...], w_ref[...], preferred_element_type=jnp.float32)
        + b_ref[...]
    )
    lo_bits = lax.bitcast_convert_type(p[:, :GP], jnp.int32) + 0x8000
    hi_bits = lax.bitcast_convert_type(p[:, GP:], jnp.int32) + 0x8000
    lo16 = lax.shift_right_logical(lo_bits, 16)
    hi16 = hi_bits & jnp.int32(-65536)
    out_ref[...] = hi16 | lo16


def _make_gather_body(rows_per_w, nchunk):
    def body(table_hbm, idx_hbm, out_hbm, idx_v, rows_v, sem):
        wid = lax.axis_index("s") * _NC + lax.axis_index("c")
        base = wid * rows_per_w
        pltpu.sync_copy(idx_hbm.at[pl.ds(base, rows_per_w)], idx_v)

        def chunk(ch, carry):
            r0 = ch * _CHUNK
            pltpu.async_copy(
                table_hbm.at[idx_v.at[pl.ds(r0, _CHUNK)]], rows_v, sem
            ).wait()
            pltpu.sync_copy(rows_v, out_hbm.at[pl.ds(base + r0, _CHUNK)])
            return carry

        lax.fori_loop(0, nchunk, chunk, 0)

    return body


def _lstm_body(x_ref, whh_ref, h_in_ref, c_in_ref, h_ref, c_ref):
    t = pl.program_id(0)

    @pl.when(t == 0)
    def _():
        h_ref[...] = h_in_ref[...]
        c_ref[...] = c_in_ref[...]

    w = x_ref[0]
    x_lo = lax.bitcast_convert_type(lax.shift_left(w, 16), jnp.float32)
    x_hi = lax.bitcast_convert_type(w & jnp.int32(-65536), jnp.float32)
    hw = jnp.dot(h_ref[...], whh_ref[...], preferred_element_type=jnp.float32)
    i = jax.nn.sigmoid(x_lo[:, 0:H] + hw[:, 0:H])
    f = jax.nn.sigmoid(x_lo[:, H:2 * H] + hw[:, H:2 * H])
    g = jnp.tanh(x_hi[:, 0:H] + hw[:, 2 * H:3 * H])
    o = jax.nn.sigmoid(x_hi[:, H:2 * H] + hw[:, 3 * H:4 * H])
    c_new = f * c_ref[...] + i * g
    h_new = o * jnp.tanh(c_new)
    c_ref[...] = c_new
    h_ref[...] = h_new.astype(jnp.bfloat16)


def _mlp_body(h_ref, w1_ref, b1_ref, w2_ref, b2_ref, out_ref):
    a = jnp.maximum(
        jnp.dot(h_ref[...], w1_ref[...], preferred_element_type=jnp.float32)
        + b1_ref[...],
        0.0,
    ).astype(jnp.bfloat16)
    out_ref[...] = (
        jnp.dot(a, w2_ref[...], preferred_element_type=jnp.float32)
        + b2_ref[...]
    )


def kernel(src_seq, src_pos, emb, W_ih, W_hh, b_ih, b_hh, W1, b1, W2, b2):
    bias = (b_ih + b_hh).reshape(1, G)
    P_packed = pl.pallas_call(
        _proj_body,
        out_shape=jax.ShapeDtypeStruct((V, GP), jnp.int32),
    )(emb, W_ih.T, bias)

    seq_t = src_seq.T.astype(jnp.int32)
    whh_bf = W_hh.T.astype(jnp.bfloat16)

    def make_gather(sz):
        rows_per_w = (sz * B) // _NW
        return pl.kernel(
            _make_gather_body(rows_per_w, rows_per_w // _CHUNK),
            out_type=jax.ShapeDtypeStruct((sz * B, GP), jnp.int32),
            mesh=plsc.VectorSubcoreMesh(core_axis_name="c",
                                        subcore_axis_name="s"),
            scratch_types=[
                pltpu.VMEM((rows_per_w,), jnp.int32),
                pltpu.VMEM((_CHUNK, GP), jnp.int32),
                pltpu.SemaphoreType.DMA,
            ],
        )

    def make_lstm(sz):
        return pl.pallas_call(
            _lstm_body,
            grid=(sz,),
            in_specs=[
                pl.BlockSpec((1, B, GP), lambda t: (t, 0, 0)),
                pl.BlockSpec((H, G), lambda t: (0, 0)),
                pl.BlockSpec((B, H), lambda t: (0, 0)),
                pl.BlockSpec((B, H), lambda t: (0, 0)),
            ],
            out_specs=[
                pl.BlockSpec((B, H), lambda t: (0, 0)),
                pl.BlockSpec((B, H), lambda t: (0, 0)),
            ],
            out_shape=[
                jax.ShapeDtypeStruct((B, H), jnp.bfloat16),
                jax.ShapeDtypeStruct((B, H), jnp.float32),
            ],
        )

    calls = {sz: (make_gather(sz), make_lstm(sz)) for sz in set(CH_SIZES)}

    h = jnp.zeros((B, H), jnp.bfloat16)
    c = jnp.zeros((B, H), jnp.float32)
    t0 = 0
    for sz in CH_SIZES:
        gather, lstm_chunk = calls[sz]
        idx_k = seq_t[t0:t0 + sz].reshape(sz * B)
        X_k = gather(P_packed, idx_k).reshape(sz, B, GP)
        h, c = lstm_chunk(X_k, whh_bf, h, c)
        t0 += sz

    W2p = jnp.pad(W2.T, ((0, 0), (0, OUT_PAD - 2))).astype(jnp.bfloat16)
    b2p = jnp.pad(b2, (0, OUT_PAD - 2)).reshape(1, OUT_PAD)
    out_p = pl.pallas_call(
        _mlp_body,
        out_shape=jax.ShapeDtypeStruct((B, OUT_PAD), jnp.float32),
    )(h, W1.T.astype(jnp.bfloat16), b1.reshape(1, F), W2p, b2p)
    return out_p[:, :2]


# speedup vs baseline: 1.0871x; 1.0141x over previous
import jax
import jax.numpy as jnp
from jax import lax
from jax.experimental import pallas as pl
from jax.experimental.pallas import tpu as pltpu
from jax.experimental.pallas import tpu_sc as plsc

B, T, V, D, H, F = 1024, 50, 1000, 512, 512, 2048
G = 4 * H
GP = G // 2
OUT_PAD = 128

CH_SIZES = (10, 10, 10, 10, 10)

_NC, _NS = 2, 16
_NW = _NC * _NS
_CHUNK = 80


def _proj_body(emb_ref, w_ref, b_ref, out_ref):
    p = (
        jnp.dot(emb_ref[...], w_ref[...], preferred_element_type=jnp.float32)
        + b_ref[...]
    )
    lo_bits = lax.bitcast_convert_type(p[:, :GP], jnp.int32) + 0x8000
    hi_bits = lax.bitcast_convert_type(p[:, GP:], jnp.int32) + 0x8000
    lo16 = lax.shift_right_logical(lo_bits, 16)
    hi16 = hi_bits & jnp.int32(-65536)
    out_ref[...] = hi16 | lo16


def _make_gather_body(rows_per_w, nchunk):
    def body(table_hbm, idx_hbm, out_hbm, idx_v, rows_v, sem):
        wid = lax.axis_index("s") * _NC + lax.axis_index("c")
        base = wid * rows_per_w
        pltpu.sync_copy(idx_hbm.at[pl.ds(base, rows_per_w)], idx_v)

        def chunk(ch, carry):
            r0 = ch * _CHUNK
            pltpu.async_copy(
                table_hbm.at[idx_v.at[pl.ds(r0, _CHUNK)]], rows_v, sem
            ).wait()
            pltpu.sync_copy(rows_v, out_hbm.at[pl.ds(base + r0, _CHUNK)])
            return carry

        lax.fori_loop(0, nchunk, chunk, 0)

    return body


def _lstm_body(x_ref, whh_ref, h_in_ref, c_in_ref, h_ref, c_ref):
    t = pl.program_id(0)

    @pl.when(t == 0)
    def _():
        h_ref[...] = h_in_ref[...]
        c_ref[...] = c_in_ref[...]

    w = x_ref[0]
    x_lo = lax.bitcast_convert_type(lax.shift_left(w, 16), jnp.float32)
    x_hi = lax.bitcast_convert_type(w & jnp.int32(-65536), jnp.float32)
    hw = jnp.dot(h_ref[...], whh_ref[...], preferred_element_type=jnp.float32)
    i = jax.nn.sigmoid(x_lo[:, 0:H] + hw[:, 0:H])
    f = jax.nn.sigmoid(x_lo[:, H:2 * H] + hw[:, H:2 * H])
    g = jnp.tanh(x_hi[:, 0:H] + hw[:, 2 * H:3 * H])
    o = jax.nn.sigmoid(x_hi[:, H:2 * H] + hw[:, 3 * H:4 * H])
    c_new = f * c_ref[...] + i * g
    h_new = o * jnp.tanh(c_new)
    c_ref[...] = c_new
    h_ref[...] = h_new.astype(jnp.bfloat16)


def _mlp_body(h_ref, w1_ref, b1_ref, w2_ref, b2_ref, out_ref):
    a = jnp.maximum(
        jnp.dot(h_ref[...], w1_ref[...], preferred_element_type=jnp.float32)
        + b1_ref[...],
        0.0,
    ).astype(jnp.bfloat16)
    out_ref[...] = (
        jnp.dot(a, w2_ref[...], preferred_element_type=jnp.float32)
        + b2_ref[...]
    )


def kernel(src_seq, src_pos, emb, W_ih, W_hh, b_ih, b_hh, W1, b1, W2, b2):
    bias = (b_ih + b_hh).reshape(1, G)
    P_packed = pl.pallas_call(
        _proj_body,
        out_shape=jax.ShapeDtypeStruct((V, GP), jnp.int32),
    )(emb, W_ih.T, bias)

    seq_t = src_seq.T.astype(jnp.int32)
    whh_bf = W_hh.T.astype(jnp.bfloat16)

    def make_gather(sz):
        rows_per_w = (sz * B) // _NW
        return pl.kernel(
            _make_gather_body(rows_per_w, rows_per_w // _CHUNK),
            out_type=jax.ShapeDtypeStruct((sz * B, GP), jnp.int32),
            mesh=plsc.VectorSubcoreMesh(core_axis_name="c",
                                        subcore_axis_name="s"),
            scratch_types=[
                pltpu.VMEM((rows_per_w,), jnp.int32),
                pltpu.VMEM((_CHUNK, GP), jnp.int32),
                pltpu.SemaphoreType.DMA,
            ],
        )

    def make_lstm(sz):
        return pl.pallas_call(
            _lstm_body,
            grid=(sz,),
            in_specs=[
                pl.BlockSpec((1, B, GP), lambda t: (t, 0, 0)),
                pl.BlockSpec((H, G), lambda t: (0, 0)),
                pl.BlockSpec((B, H), lambda t: (0, 0)),
                pl.BlockSpec((B, H), lambda t: (0, 0)),
            ],
            out_specs=[
                pl.BlockSpec((B, H), lambda t: (0, 0)),
                pl.BlockSpec((B, H), lambda t: (0, 0)),
            ],
            out_shape=[
                jax.ShapeDtypeStruct((B, H), jnp.bfloat16),
                jax.ShapeDtypeStruct((B, H), jnp.float32),
            ],
        )

    calls = {sz: (make_gather(sz), make_lstm(sz)) for sz in set(CH_SIZES)}

    h = jnp.zeros((B, H), jnp.bfloat16)
    c = jnp.zeros((B, H), jnp.float32)
    t0 = 0
    for sz in CH_SIZES:
        gather, lstm_chunk = calls[sz]
        idx_k = seq_t[t0:t0 + sz].reshape(sz * B)
        X_k = gather(P_packed, idx_k).reshape(sz, B, GP)
        h, c = lstm_chunk(X_k, whh_bf, h, c)
        t0 += sz

    W2p = jnp.pad(W2.T, ((0, 0), (0, OUT_PAD - 2))).astype(jnp.bfloat16)
    b2p = jnp.pad(b2, (0, OUT_PAD - 2)).reshape(1, OUT_PAD)
    out_p = pl.pallas_call(
        _mlp_body,
        out_shape=jax.ShapeDtypeStruct((B, OUT_PAD), jnp.float32),
    )(h, W1.T.astype(jnp.bfloat16), b1.reshape(1, F), W2p, b2p)
    return out_p[:, :2]
